# baseline (speedup 1.0000x reference)
import jax
import jax.numpy as jnp
import numpy as np
from jax.experimental import pallas as pl

N = 50000
E = 800000
MS = 32
MV = 16
NB = 8
L = 4
EMB = 8
NIN = 2
MAX_R = 5.0
NUM_NEIGH = 16.0


def _semi_unitary(M):
    I = jnp.eye(M.shape[0], dtype=M.dtype)
    for _ in range(10):
        M = M - 0.5 * (M @ M.T - I) @ M
    return M


def kernel(x, node_attr, edge_src, edge_dst, emb, M_proj, Wr1, br1, Wr2, Wvs, Wsv, Wgate, Wemb, Us, Uv):
    n = x.shape[0]
    r = x[:, -3:]
    edge_vec = r[edge_dst] - r[edge_src]
    edge_len = jnp.sqrt(jnp.sum(edge_vec ** 2, axis=-1) + 1e-12)
    sh1 = edge_vec / edge_len[:, None]
    centers = jnp.linspace(0.0, MAX_R, NB)
    width = MAX_R / NB
    basis = jnp.exp(-(((edge_len[:, None] - centers) / width) ** 2))
    cutoff = 0.5 * (jnp.cos(jnp.pi * jnp.clip(edge_len / MAX_R, 0.0, 1.0)) + 1.0)
    M = _semi_unitary(M_proj)
    x2 = x.reshape(n, -1, 3).transpose(0, 2, 1)
    v = (x2 @ M).transpose(0, 2, 1)
    s = jnp.zeros((n, MS), dtype=x.dtype)
    z = emb[node_attr]
    for l in range(L):
        h = jax.nn.silu(basis @ Wr1[l] + br1[l])
        w = (h @ Wr2[l]) * cutoff[:, None]
        w_ss = w[:, :MS]
        w_vs = w[:, MS:MS + MV]
        w_vv = w[:, MS + MV:MS + 2 * MV]
        w_sv = w[:, MS + 2 * MV:]
        s_src = s[edge_src]
        v_src = v[edge_src]
        dot = jnp.einsum('evc,ec->ev', v_src, sh1)
        msg_s = s_src * w_ss + (dot * w_vs) @ Wvs[l]
        msg_v = v_src * w_vv[:, :, None] + ((s_src @ Wsv[l]) * w_sv)[:, :, None] * sh1[:, None, :]
        s_agg = jax.ops.segment_sum(msg_s, edge_dst, num_segments=n) / jnp.sqrt(NUM_NEIGH)
        v_agg = jax.ops.segment_sum(msg_v, edge_dst, num_segments=n) / jnp.sqrt(NUM_NEIGH)
        s_agg = s_agg + z @ Wemb[l]
        gates = jax.nn.sigmoid(s_agg @ Wgate[l])
        s = jax.nn.silu(s_agg) @ Us[l]
        v = jnp.einsum('nvc,vw->nwc', v_agg * gates[:, :, None], Uv[l])
    y2 = v.transpose(0, 2, 1)
    out = (y2 @ M.T).transpose(0, 2, 1).reshape(n, -1)
    return out


# speedup vs baseline: 7.8489x; 7.8489x over previous
import functools

import jax
import jax.numpy as jnp
from jax import lax
from jax.experimental import pallas as pl
from jax.experimental.pallas import tpu as pltpu
from jax.experimental.pallas import tpu_sc as plsc

F32 = jnp.float32

MS = 32
MV = 16
NB = 8
LAYERS = 4
EMBW = 8
MAX_R = 5.0
INV_SQRT_NEIGH = 0.25

NC = 2
NS = 16
NW = NC * NS
CH = 128
NP = 50048

BE = 2000
BN = 2000


def _semi_unitary_jnp(M):
    I = jnp.eye(M.shape[0], dtype=M.dtype)
    for _ in range(10):
        M = M - 0.5 * (M @ M.T - I) @ M
    return M


def _mesh():
    return plsc.VectorSubcoreMesh(core_axis_name="c", subcore_axis_name="s",
                                  num_cores=NC, num_subcores=NS)


def _sc_gather(tab, idx):
    B = idx.shape[0]
    per_w = B // NW
    nch, tail = divmod(per_w, CH)
    scratch = [
        pltpu.VMEM((CH,), jnp.int32),
        pltpu.VMEM((CH, 128), F32),
        pltpu.SemaphoreType.DMA,
    ]
    if tail:
        scratch += [pltpu.VMEM((tail,), jnp.int32), pltpu.VMEM((tail, 128), F32)]

    @functools.partial(
        pl.kernel,
        out_type=jax.ShapeDtypeStruct((B, 128), F32),
        mesh=_mesh(),
        scratch_types=scratch,
    )
    def gk(tab_ref, idx_ref, out_ref, idxv, rows, sem, *tails):
        wid = lax.axis_index("s") * NC + lax.axis_index("c")
        base = pl.multiple_of(wid * per_w, 8)

        @pl.loop(0, nch)
        def chunk(i):
            b = pl.multiple_of(base + i * CH, 8)
            pltpu.sync_copy(idx_ref.at[pl.ds(b, CH)], idxv)
            pltpu.async_copy(tab_ref.at[idxv], rows, sem).wait()
            pltpu.sync_copy(rows, out_ref.at[pl.ds(b, CH)])

        if tail:
            idxt, rowst = tails
            b = pl.multiple_of(base + nch * CH, 8)
            pltpu.sync_copy(idx_ref.at[pl.ds(b, tail)], idxt)
            pltpu.async_copy(tab_ref.at[idxt], rowst, sem).wait()
            pltpu.sync_copy(rowst, out_ref.at[pl.ds(b, tail)])

    return gk(tab, idx)


def _full(shape):
    nd = len(shape)
    return pl.BlockSpec(shape, lambda i: (0,) * nd)


def _tc_init(x, attr_col, Mbig, emb, N):
    BI = 1000
    grid = N // BI

    def body(x_ref, a_ref, mb_ref, emb_ref, nt_ref, z_ref):
        xb = x_ref[...]
        vt = jnp.dot(xb, mb_ref[...], preferred_element_type=F32)
        nt_ref[...] = jnp.concatenate(
            [jnp.zeros((BI, 48), F32), vt, xb[:, 3:6],
             jnp.zeros((BI, 29), F32)], axis=1)
        oh = jnp.where(
            a_ref[...] == lax.broadcasted_iota(jnp.int32, (BI, 20), 1).astype(F32),
            1.0, 0.0)
        z_ref[...] = jnp.dot(oh, emb_ref[...], preferred_element_type=F32)

    return pl.pallas_call(
        body,
        grid=(grid,),
        in_specs=[
            pl.BlockSpec((BI, 6), lambda i: (i, 0)),
            pl.BlockSpec((BI, 1), lambda i: (i, 0)),
            _full((6, 48)),
            _full((20, EMBW)),
        ],
        out_specs=[
            pl.BlockSpec((BI, 128), lambda i: (i, 0)),
            pl.BlockSpec((BI, EMBW), lambda i: (i, 0)),
        ],
        out_shape=[
            jax.ShapeDtypeStruct((N, 128), F32),
            jax.ShapeDtypeStruct((N, EMBW), F32),
        ],
    )(x, attr_col, Mbig, emb)


def _tc_geo(RR, E):
    grid = E // BE

    def body(rs_ref, rd_ref, geo_ref):
        ev = rd_ref[...][:, 96:99] - rs_ref[...][:, 96:99]
        l2 = jnp.sum(ev * ev, axis=1, keepdims=True) + 1e-12
        el = jnp.sqrt(l2)
        sh = ev / el
        centers = lax.broadcasted_iota(jnp.int32, (1, NB), 1).astype(F32) * (
            MAX_R / (NB - 1))
        width = MAX_R / NB
        basis = jnp.exp(-(((el - centers) / width) ** 2))
        cut = 0.5 * (jnp.cos(jnp.pi * jnp.clip(el / MAX_R, 0.0, 1.0)) + 1.0)
        geo_ref[...] = jnp.concatenate(
            [basis, sh, cut, jnp.zeros((BE, 4), F32)], axis=1)

    return pl.pallas_call(
        body,
        grid=(grid,),
        in_specs=[
            pl.BlockSpec((BE, 128), lambda i: (i, 0)),
            pl.BlockSpec((BE, 128), lambda i: (i + grid, 0)),
        ],
        out_specs=pl.BlockSpec((BE, 16), lambda i: (i, 0)),
        out_shape=jax.ShapeDtypeStruct((E, 16), F32),
    )(RR, RR)


def _tc_edge(geo, G, Wr1l, br1l, Wr2l, E):
    grid = E // BE

    def body(geo_ref, g_ref, w1_ref, b1_ref, w2_ref,
             m0_ref, m1_ref, m2_ref, m3_ref, m4_ref, m5_ref):
        geo = geo_ref[...]
        G = g_ref[...]
        basis = geo[:, 0:8]
        h = jnp.dot(basis, w1_ref[...], preferred_element_type=F32) + b1_ref[...]
        h = h * jax.nn.sigmoid(h)
        w = jnp.dot(h, w2_ref[...], preferred_element_type=F32)
        w = w * geo[:, 11:12]
        sh0 = geo[:, 8:9]
        sh1 = geo[:, 9:10]
        sh2 = geo[:, 10:11]
        dot = G[:, 48:64] * sh0 + G[:, 64:80] * sh1 + G[:, 80:96] * sh2
        wvv = w[:, 48:64]
        t = G[:, 32:48] * w[:, 64:80]
        msg0 = G[:, 0:32] * w[:, 0:32]
        msgB = dot * w[:, 32:48]
        mc0 = G[:, 48:64] * wvv + t * sh0
        mc1 = G[:, 64:80] * wvv + t * sh1
        mc2 = G[:, 80:96] * wvv + t * sh2
        m0_ref[...] = msg0[:, 0:16]
        m1_ref[...] = msg0[:, 16:32]
        m2_ref[...] = msgB
        m3_ref[...] = mc0
        m4_ref[...] = mc1
        m5_ref[...] = mc2

    part_spec = pl.BlockSpec((BE, 16), lambda i: (i, 0))
    part_shape = jax.ShapeDtypeStruct((E, 16), F32)
    return pl.pallas_call(
        body,
        grid=(grid,),
        in_specs=[
            pl.BlockSpec((BE, 16), lambda i: (i, 0)),
            pl.BlockSpec((BE, 128), lambda i: (i, 0)),
            _full((NB, 16)),
            _full((1, 16)),
            _full((16, MS + 3 * MV)),
        ],
        out_specs=[part_spec] * 6,
        out_shape=[part_shape] * 6,
    )(geo, G, Wr1l, br1l, Wr2l)


def _tc_node(accP, z, Wvsl, Wembl, Wgatel, Usl, Uvl, last, N, final):
    grid = N // BN

    def body(aP_ref, z_ref, wvs_ref, wemb_ref, wgate_ref, us_ref,
             uv_ref, last_ref, out_ref):
        aP = aP_ref[...] * INV_SQRT_NEIGH
        s_raw = jnp.concatenate([aP[0], aP[1]], axis=1)
        aB = aP[2]
        s_agg = s_raw + jnp.dot(aB, wvs_ref[...], preferred_element_type=F32)
        s_agg = s_agg + jnp.dot(z_ref[...], wemb_ref[...],
                                preferred_element_type=F32)
        gates = jax.nn.sigmoid(jnp.dot(s_agg, wgate_ref[...],
                                       preferred_element_type=F32))
        s_new = jnp.dot(s_agg * jax.nn.sigmoid(s_agg), us_ref[...],
                        preferred_element_type=F32)
        uv = uv_ref[...]
        vagg0 = aP[3]
        vagg1 = aP[4]
        vagg2 = aP[5]
        vn0 = jnp.dot(vagg0 * gates, uv, preferred_element_type=F32)
        vn1 = jnp.dot(vagg1 * gates, uv, preferred_element_type=F32)
        vn2 = jnp.dot(vagg2 * gates, uv, preferred_element_type=F32)
        vn = jnp.concatenate([vn0, vn1, vn2], axis=1)
        if final:
            out_ref[...] = jnp.dot(vn, last_ref[...],
                                   preferred_element_type=F32)
        else:
            sw = jnp.dot(s_new, last_ref[...], preferred_element_type=F32)
            out_ref[...] = jnp.concatenate(
                [s_new, sw, vn, jnp.zeros((BN, 32), F32)], axis=1)

    ow = 6 if final else 128
    return pl.pallas_call(
        body,
        grid=(grid,),
        in_specs=[
            pl.BlockSpec((6, BN, 16), lambda i: (0, i, 0)),
            pl.BlockSpec((BN, EMBW), lambda i: (i, 0)),
            _full((MV, MS)),
            _full((EMBW, MS)),
            _full((MS, MV)),
            _full((MS, MS)),
            _full((MV, MV)),
            _full(last.shape),
        ],
        out_specs=pl.BlockSpec((BN, ow), lambda i: (i, 0)),
        out_shape=jax.ShapeDtypeStruct((N, ow), F32),
    )(accP, z, Wvsl, Wembl, Wgatel, Usl, Uvl, last)


def kernel(x, node_attr, edge_src, edge_dst, emb, M_proj, Wr1, br1, Wr2,
           Wvs, Wsv, Wgate, Wemb, Us, Uv):
    N = x.shape[0]
    E = edge_src.shape[0]

    M = _semi_unitary_jnp(M_proj)
    Mbig = jnp.zeros((6, 48), F32)
    MbigT = jnp.zeros((48, 6), F32)
    for i in range(2):
        for c in range(3):
            Mbig = Mbig.at[i * 3 + c, c * 16:(c + 1) * 16].set(M[i])
            MbigT = MbigT.at[c * 16:(c + 1) * 16, i * 3 + c].set(M[i])

    attr_col = node_attr.astype(F32).reshape(N, 1)
    esrc = edge_src.astype(jnp.int32)
    edst = edge_dst.astype(jnp.int32)

    nt, z = _tc_init(x, attr_col, Mbig, emb, N)

    RR = _sc_gather(nt, jnp.concatenate([esrc, edst]))
    geo = _tc_geo(RR, E)


    out = None
    for l in range(LAYERS):
        G = RR if l == 0 else _sc_gather(nt, esrc)
        ms = _tc_edge(geo, G, Wr1[l], br1[l].reshape(1, 16), Wr2[l], E)
        accP = jnp.stack([jax.ops.segment_sum(m, edst, num_segments=NP)
                          for m in ms])
        final = l == LAYERS - 1
        last = MbigT if final else Wsv[l + 1]
        res = _tc_node(accP, z, Wvs[l], Wemb[l], Wgate[l], Us[l], Uv[l],
                       last, N, final)
        if final:
            out = res
        else:
            nt = res
    return out


# speedup vs baseline: 20.2984x; 2.5861x over previous
import functools

import jax
import jax.numpy as jnp
from jax import lax
from jax.experimental import pallas as pl
from jax.experimental.pallas import tpu as pltpu
from jax.experimental.pallas import tpu_sc as plsc

F32 = jnp.float32

MS = 32
MV = 16
NB = 8
LAYERS = 4
EMBW = 8
MAX_R = 5.0
INV_SQRT_NEIGH = 0.25

NC = 2
NS = 16
NW = NC * NS
CH = 128
NP = 50048

BE = 2000
BN = 2000


def _semi_unitary_jnp(M):
    I = jnp.eye(M.shape[0], dtype=M.dtype)
    for _ in range(10):
        M = M - 0.5 * (M @ M.T - I) @ M
    return M


def _mesh():
    return plsc.VectorSubcoreMesh(core_axis_name="c", subcore_axis_name="s",
                                  num_cores=NC, num_subcores=NS)


def _sc_gather(tab, idx):
    B = idx.shape[0]
    per_w = B // NW
    nch, tail = divmod(per_w, CH)
    scratch = [
        pltpu.VMEM((CH,), jnp.int32),
        pltpu.VMEM((CH, 128), F32),
        pltpu.SemaphoreType.DMA,
    ]
    if tail:
        scratch += [pltpu.VMEM((tail,), jnp.int32), pltpu.VMEM((tail, 128), F32)]

    @functools.partial(
        pl.kernel,
        out_type=jax.ShapeDtypeStruct((B, 128), F32),
        mesh=_mesh(),
        scratch_types=scratch,
    )
    def gk(tab_ref, idx_ref, out_ref, idxv, rows, sem, *tails):
        wid = lax.axis_index("s") * NC + lax.axis_index("c")
        base = pl.multiple_of(wid * per_w, 8)

        @pl.loop(0, nch)
        def chunk(i):
            b = pl.multiple_of(base + i * CH, 8)
            pltpu.sync_copy(idx_ref.at[pl.ds(b, CH)], idxv)
            pltpu.async_copy(tab_ref.at[idxv], rows, sem).wait()
            pltpu.sync_copy(rows, out_ref.at[pl.ds(b, CH)])

        if tail:
            idxt, rowst = tails
            b = pl.multiple_of(base + nch * CH, 8)
            pltpu.sync_copy(idx_ref.at[pl.ds(b, tail)], idxt)
            pltpu.async_copy(tab_ref.at[idxt], rowst, sem).wait()
            pltpu.sync_copy(rowst, out_ref.at[pl.ds(b, tail)])

    return gk(tab, idx)


def _full(shape):
    nd = len(shape)
    return pl.BlockSpec(shape, lambda i: (0,) * nd)


def _tc_init(x, attr_col, Mbig, emb, N):
    BI = 1000
    grid = N // BI

    def body(x_ref, a_ref, mb_ref, emb_ref, nt_ref, z_ref):
        xb = x_ref[...]
        vt = jnp.dot(xb, mb_ref[...], preferred_element_type=F32)
        nt_ref[...] = jnp.concatenate(
            [jnp.zeros((BI, 48), F32), vt, xb[:, 3:6],
             jnp.zeros((BI, 29), F32)], axis=1)
        oh = jnp.where(
            a_ref[...] == lax.broadcasted_iota(jnp.int32, (BI, 20), 1).astype(F32),
            1.0, 0.0)
        z_ref[...] = jnp.dot(oh, emb_ref[...], preferred_element_type=F32)

    return pl.pallas_call(
        body,
        grid=(grid,),
        in_specs=[
            pl.BlockSpec((BI, 6), lambda i: (i, 0)),
            pl.BlockSpec((BI, 1), lambda i: (i, 0)),
            _full((6, 48)),
            _full((20, EMBW)),
        ],
        out_specs=[
            pl.BlockSpec((BI, 128), lambda i: (i, 0)),
            pl.BlockSpec((BI, EMBW), lambda i: (i, 0)),
        ],
        out_shape=[
            jax.ShapeDtypeStruct((N, 128), F32),
            jax.ShapeDtypeStruct((N, EMBW), F32),
        ],
    )(x, attr_col, Mbig, emb)


def _tc_geo(RR, E):
    grid = E // BE

    def body(rs_ref, rd_ref, geo_ref):
        ev = rd_ref[...][:, 96:99] - rs_ref[...][:, 96:99]
        l2 = jnp.sum(ev * ev, axis=1, keepdims=True) + 1e-12
        el = jnp.sqrt(l2)
        sh = ev / el
        centers = lax.broadcasted_iota(jnp.int32, (1, NB), 1).astype(F32) * (
            MAX_R / (NB - 1))
        width = MAX_R / NB
        basis = jnp.exp(-(((el - centers) / width) ** 2))
        cut = 0.5 * (jnp.cos(jnp.pi * jnp.clip(el / MAX_R, 0.0, 1.0)) + 1.0)
        geo_ref[...] = jnp.concatenate(
            [basis, sh, cut, jnp.zeros((BE, 4), F32)], axis=1)

    return pl.pallas_call(
        body,
        grid=(grid,),
        in_specs=[
            pl.BlockSpec((BE, 128), lambda i: (i, 0)),
            pl.BlockSpec((BE, 128), lambda i: (i + grid, 0)),
        ],
        out_specs=pl.BlockSpec((BE, 16), lambda i: (i, 0)),
        out_shape=jax.ShapeDtypeStruct((E, 16), F32),
    )(RR, RR)


def _tc_edge(geo, G, Wr1l, br1l, Wr2l, E):
    grid = E // BE

    def body(geo_ref, g_ref, w1_ref, b1_ref, w2_ref, m_ref):
        geo = geo_ref[...]
        G = g_ref[...]
        basis = geo[:, 0:8]
        h = jnp.dot(basis, w1_ref[...], preferred_element_type=F32) + b1_ref[...]
        h = h * jax.nn.sigmoid(h)
        w = jnp.dot(h, w2_ref[...], preferred_element_type=F32)
        w = w * geo[:, 11:12]
        sh0 = geo[:, 8:9]
        sh1 = geo[:, 9:10]
        sh2 = geo[:, 10:11]
        dot = G[:, 48:64] * sh0 + G[:, 64:80] * sh1 + G[:, 80:96] * sh2
        wvv = w[:, 48:64]
        t = G[:, 32:48] * w[:, 64:80]
        msg0 = G[:, 0:32] * w[:, 0:32]
        msgB = dot * w[:, 32:48]
        mc0 = G[:, 48:64] * wvv + t * sh0
        mc1 = G[:, 64:80] * wvv + t * sh1
        mc2 = G[:, 80:96] * wvv + t * sh2
        m_ref[...] = jnp.concatenate([msg0, msgB, mc0, mc1, mc2], axis=1)

    part_spec = pl.BlockSpec((BE, 96), lambda i: (i, 0))
    part_shape = jax.ShapeDtypeStruct((E, 96), F32)
    return pl.pallas_call(
        body,
        grid=(grid,),
        in_specs=[
            pl.BlockSpec((BE, 16), lambda i: (i, 0)),
            pl.BlockSpec((BE, 128), lambda i: (i, 0)),
            _full((NB, 16)),
            _full((1, 16)),
            _full((16, MS + 3 * MV)),
        ],
        out_specs=part_spec,
        out_shape=part_shape,
    )(geo, G, Wr1l, br1l, Wr2l)


def _tc_node(accP, z, Wvsl, Wembl, Wgatel, Usl, Uvl, last, N, final):
    grid = N // BN

    def body(aP_ref, z_ref, wvs_ref, wemb_ref, wgate_ref, us_ref,
             uv_ref, last_ref, out_ref):
        aP = aP_ref[...] * INV_SQRT_NEIGH
        s_raw = aP[:, 0:32]
        aB = aP[:, 32:48]
        s_agg = s_raw + jnp.dot(aB, wvs_ref[...], preferred_element_type=F32)
        s_agg = s_agg + jnp.dot(z_ref[...], wemb_ref[...],
                                preferred_element_type=F32)
        gates = jax.nn.sigmoid(jnp.dot(s_agg, wgate_ref[...],
                                       preferred_element_type=F32))
        s_new = jnp.dot(s_agg * jax.nn.sigmoid(s_agg), us_ref[...],
                        preferred_element_type=F32)
        uv = uv_ref[...]
        vagg0 = aP[:, 48:64]
        vagg1 = aP[:, 64:80]
        vagg2 = aP[:, 80:96]
        vn0 = jnp.dot(vagg0 * gates, uv, preferred_element_type=F32)
        vn1 = jnp.dot(vagg1 * gates, uv, preferred_element_type=F32)
        vn2 = jnp.dot(vagg2 * gates, uv, preferred_element_type=F32)
        vn = jnp.concatenate([vn0, vn1, vn2], axis=1)
        if final:
            out_ref[...] = jnp.dot(vn, last_ref[...],
                                   preferred_element_type=F32)
        else:
            sw = jnp.dot(s_new, last_ref[...], preferred_element_type=F32)
            out_ref[...] = jnp.concatenate(
                [s_new, sw, vn, jnp.zeros((BN, 32), F32)], axis=1)

    ow = 6 if final else 128
    return pl.pallas_call(
        body,
        grid=(grid,),
        in_specs=[
            pl.BlockSpec((BN, 96), lambda i: (i, 0)),
            pl.BlockSpec((BN, EMBW), lambda i: (i, 0)),
            _full((MV, MS)),
            _full((EMBW, MS)),
            _full((MS, MV)),
            _full((MS, MS)),
            _full((MV, MV)),
            _full(last.shape),
        ],
        out_specs=pl.BlockSpec((BN, ow), lambda i: (i, 0)),
        out_shape=jax.ShapeDtypeStruct((N, ow), F32),
    )(accP, z, Wvsl, Wembl, Wgatel, Usl, Uvl, last)


def kernel(x, node_attr, edge_src, edge_dst, emb, M_proj, Wr1, br1, Wr2,
           Wvs, Wsv, Wgate, Wemb, Us, Uv):
    N = x.shape[0]
    E = edge_src.shape[0]

    M = _semi_unitary_jnp(M_proj)
    Mbig = jnp.zeros((6, 48), F32)
    MbigT = jnp.zeros((48, 6), F32)
    for i in range(2):
        for c in range(3):
            Mbig = Mbig.at[i * 3 + c, c * 16:(c + 1) * 16].set(M[i])
            MbigT = MbigT.at[c * 16:(c + 1) * 16, i * 3 + c].set(M[i])

    attr_col = node_attr.astype(F32).reshape(N, 1)
    esrc = edge_src.astype(jnp.int32)
    edst = edge_dst.astype(jnp.int32)

    nt, z = _tc_init(x, attr_col, Mbig, emb, N)

    RR = _sc_gather(nt, jnp.concatenate([esrc, edst]))
    geo = _tc_geo(RR, E)


    out = None
    for l in range(LAYERS):
        G = RR if l == 0 else _sc_gather(nt, esrc)
        m96 = _tc_edge(geo, G, Wr1[l], br1[l].reshape(1, 16), Wr2[l], E)
        accP = jax.ops.segment_sum(m96, edst, num_segments=NP)
        final = l == LAYERS - 1
        last = MbigT if final else Wsv[l + 1]
        res = _tc_node(accP, z, Wvs[l], Wemb[l], Wgate[l], Us[l], Uv[l],
                       last, N, final)
        if final:
            out = res
        else:
            nt = res
    return out


# speedup vs baseline: 21.1872x; 1.0438x over previous
import functools

import jax
import jax.numpy as jnp
from jax import lax
from jax.experimental import pallas as pl
from jax.experimental.pallas import tpu as pltpu
from jax.experimental.pallas import tpu_sc as plsc

F32 = jnp.float32

MS = 32
MV = 16
NB = 8
LAYERS = 4
EMBW = 8
MAX_R = 5.0
INV_SQRT_NEIGH = 0.25

NC = 2
NS = 16
NW = NC * NS
CH = 128
NP = 50048

BE = 2000
BN = 2000


def _semi_unitary_jnp(M):
    I = jnp.eye(M.shape[0], dtype=M.dtype)
    for _ in range(10):
        M = M - 0.5 * (M @ M.T - I) @ M
    return M


def _mesh():
    return plsc.VectorSubcoreMesh(core_axis_name="c", subcore_axis_name="s",
                                  num_cores=NC, num_subcores=NS)


def _sc_gather(tab, idx):
    B = idx.shape[0]
    per_w = B // NW
    nch, tail = divmod(per_w, CH)
    npair = nch // 2
    odd = nch % 2
    scratch = [
        pltpu.VMEM((CH,), jnp.int32),
        pltpu.VMEM((CH,), jnp.int32),
        pltpu.VMEM((CH, 128), F32),
        pltpu.VMEM((CH, 128), F32),
        pltpu.SemaphoreType.DMA,
        pltpu.SemaphoreType.DMA,
    ]
    if tail:
        scratch += [pltpu.VMEM((tail,), jnp.int32), pltpu.VMEM((tail, 128), F32)]

    @functools.partial(
        pl.kernel,
        out_type=jax.ShapeDtypeStruct((B, 128), F32),
        mesh=_mesh(),
        scratch_types=scratch,
    )
    def gk(tab_ref, idx_ref, out_ref, idxA, idxB, rowsA, rowsB, semA, semB,
           *tails):
        wid = lax.axis_index("s") * NC + lax.axis_index("c")
        base = pl.multiple_of(wid * per_w, 8)
        last_b = B - CH

        pltpu.sync_copy(idx_ref.at[pl.ds(base, CH)], idxA)
        pltpu.async_copy(tab_ref.at[idxA], rowsA, semA)

        @pl.loop(0, npair)
        def pair(k):
            i0 = 2 * k
            b0 = pl.multiple_of(base + i0 * CH, 8)
            b1 = pl.multiple_of(base + (i0 + 1) * CH, 8)
            pltpu.sync_copy(idx_ref.at[pl.ds(b1, CH)], idxB)
            pltpu.async_copy(tab_ref.at[idxB], rowsB, semB)
            pltpu.make_async_copy(tab_ref.at[idxA], rowsA, semA).wait()
            pltpu.sync_copy(rowsA, out_ref.at[pl.ds(b0, CH)])
            b2 = pl.multiple_of(
                jnp.minimum(base + (i0 + 2) * CH, last_b), 8)
            pltpu.sync_copy(idx_ref.at[pl.ds(b2, CH)], idxA)
            pltpu.async_copy(tab_ref.at[idxA], rowsA, semA)
            pltpu.make_async_copy(tab_ref.at[idxB], rowsB, semB).wait()
            pltpu.sync_copy(rowsB, out_ref.at[pl.ds(b1, CH)])

        pltpu.make_async_copy(tab_ref.at[idxA], rowsA, semA).wait()
        if odd:
            b = pl.multiple_of(base + (nch - 1) * CH, 8)
            pltpu.sync_copy(rowsA, out_ref.at[pl.ds(b, CH)])

        if tail:
            idxt, rowst = tails
            b = pl.multiple_of(base + nch * CH, 8)
            pltpu.sync_copy(idx_ref.at[pl.ds(b, tail)], idxt)
            pltpu.async_copy(tab_ref.at[idxt], rowst, semA).wait()
            pltpu.sync_copy(rowst, out_ref.at[pl.ds(b, tail)])

    return gk(tab, idx)


def _full(shape):
    nd = len(shape)
    return pl.BlockSpec(shape, lambda i: (0,) * nd)


def _tc_init(x, attr_col, Mbig, emb, N):
    BI = 1000
    grid = N // BI

    def body(x_ref, a_ref, mb_ref, emb_ref, nt_ref, z_ref):
        xb = x_ref[...]
        vt = jnp.dot(xb, mb_ref[...], preferred_element_type=F32)
        nt_ref[...] = jnp.concatenate(
            [jnp.zeros((BI, 48), F32), vt, xb[:, 3:6],
             jnp.zeros((BI, 29), F32)], axis=1)
        oh = jnp.where(
            a_ref[...] == lax.broadcasted_iota(jnp.int32, (BI, 20), 1).astype(F32),
            1.0, 0.0)
        z_ref[...] = jnp.dot(oh, emb_ref[...], preferred_element_type=F32)

    return pl.pallas_call(
        body,
        grid=(grid,),
        in_specs=[
            pl.BlockSpec((BI, 6), lambda i: (i, 0)),
            pl.BlockSpec((BI, 1), lambda i: (i, 0)),
            _full((6, 48)),
            _full((20, EMBW)),
        ],
        out_specs=[
            pl.BlockSpec((BI, 128), lambda i: (i, 0)),
            pl.BlockSpec((BI, EMBW), lambda i: (i, 0)),
        ],
        out_shape=[
            jax.ShapeDtypeStruct((N, 128), F32),
            jax.ShapeDtypeStruct((N, EMBW), F32),
        ],
    )(x, attr_col, Mbig, emb)


def _tc_geo(RR, E):
    grid = E // BE

    def body(rs_ref, rd_ref, geo_ref):
        ev = rd_ref[...][:, 96:99] - rs_ref[...][:, 96:99]
        l2 = jnp.sum(ev * ev, axis=1, keepdims=True) + 1e-12
        el = jnp.sqrt(l2)
        sh = ev / el
        centers = lax.broadcasted_iota(jnp.int32, (1, NB), 1).astype(F32) * (
            MAX_R / (NB - 1))
        width = MAX_R / NB
        basis = jnp.exp(-(((el - centers) / width) ** 2))
        cut = 0.5 * (jnp.cos(jnp.pi * jnp.clip(el / MAX_R, 0.0, 1.0)) + 1.0)
        geo_ref[...] = jnp.concatenate(
            [basis, sh, cut, jnp.zeros((BE, 4), F32)], axis=1)

    return pl.pallas_call(
        body,
        grid=(grid,),
        in_specs=[
            pl.BlockSpec((BE, 128), lambda i: (i, 0)),
            pl.BlockSpec((BE, 128), lambda i: (i + grid, 0)),
        ],
        out_specs=pl.BlockSpec((BE, 16), lambda i: (i, 0)),
        out_shape=jax.ShapeDtypeStruct((E, 16), F32),
    )(RR, RR)


def _tc_edge(geo, G, Wr1l, br1l, Wr2l, E):
    grid = E // BE

    def body(geo_ref, g_ref, w1_ref, b1_ref, w2_ref, m_ref):
        geo = geo_ref[...]
        G = g_ref[...]
        basis = geo[:, 0:8]
        h = jnp.dot(basis, w1_ref[...], preferred_element_type=F32) + b1_ref[...]
        h = h * jax.nn.sigmoid(h)
        w = jnp.dot(h, w2_ref[...], preferred_element_type=F32)
        w = w * geo[:, 11:12]
        sh0 = geo[:, 8:9]
        sh1 = geo[:, 9:10]
        sh2 = geo[:, 10:11]
        dot = G[:, 48:64] * sh0 + G[:, 64:80] * sh1 + G[:, 80:96] * sh2
        wvv = w[:, 48:64]
        t = G[:, 32:48] * w[:, 64:80]
        msg0 = G[:, 0:32] * w[:, 0:32]
        msgB = dot * w[:, 32:48]
        mc0 = G[:, 48:64] * wvv + t * sh0
        mc1 = G[:, 64:80] * wvv + t * sh1
        mc2 = G[:, 80:96] * wvv + t * sh2
        m_ref[...] = jnp.concatenate([msg0, msgB, mc0, mc1, mc2], axis=1)

    part_spec = pl.BlockSpec((BE, 96), lambda i: (i, 0))
    part_shape = jax.ShapeDtypeStruct((E, 96), F32)
    return pl.pallas_call(
        body,
        grid=(grid,),
        in_specs=[
            pl.BlockSpec((BE, 16), lambda i: (i, 0)),
            pl.BlockSpec((BE, 128), lambda i: (i, 0)),
            _full((NB, 16)),
            _full((1, 16)),
            _full((16, MS + 3 * MV)),
        ],
        out_specs=part_spec,
        out_shape=part_shape,
    )(geo, G, Wr1l, br1l, Wr2l)


def _tc_node(accP, z, Wvsl, Wembl, Wgatel, Usl, Uvl, last, N, final):
    grid = N // BN

    def body(aP_ref, z_ref, wvs_ref, wemb_ref, wgate_ref, us_ref,
             uv_ref, last_ref, out_ref):
        aP = aP_ref[...] * INV_SQRT_NEIGH
        s_raw = aP[:, 0:32]
        aB = aP[:, 32:48]
        s_agg = s_raw + jnp.dot(aB, wvs_ref[...], preferred_element_type=F32)
        s_agg = s_agg + jnp.dot(z_ref[...], wemb_ref[...],
                                preferred_element_type=F32)
        gates = jax.nn.sigmoid(jnp.dot(s_agg, wgate_ref[...],
                                       preferred_element_type=F32))
        s_new = jnp.dot(s_agg * jax.nn.sigmoid(s_agg), us_ref[...],
                        preferred_element_type=F32)
        uv = uv_ref[...]
        vagg0 = aP[:, 48:64]
        vagg1 = aP[:, 64:80]
        vagg2 = aP[:, 80:96]
        vn0 = jnp.dot(vagg0 * gates, uv, preferred_element_type=F32)
        vn1 = jnp.dot(vagg1 * gates, uv, preferred_element_type=F32)
        vn2 = jnp.dot(vagg2 * gates, uv, preferred_element_type=F32)
        vn = jnp.concatenate([vn0, vn1, vn2], axis=1)
        if final:
            out_ref[...] = jnp.dot(vn, last_ref[...],
                                   preferred_element_type=F32)
        else:
            sw = jnp.dot(s_new, last_ref[...], preferred_element_type=F32)
            out_ref[...] = jnp.concatenate(
                [s_new, sw, vn, jnp.zeros((BN, 32), F32)], axis=1)

    ow = 6 if final else 128
    return pl.pallas_call(
        body,
        grid=(grid,),
        in_specs=[
            pl.BlockSpec((BN, 96), lambda i: (i, 0)),
            pl.BlockSpec((BN, EMBW), lambda i: (i, 0)),
            _full((MV, MS)),
            _full((EMBW, MS)),
            _full((MS, MV)),
            _full((MS, MS)),
            _full((MV, MV)),
            _full(last.shape),
        ],
        out_specs=pl.BlockSpec((BN, ow), lambda i: (i, 0)),
        out_shape=jax.ShapeDtypeStruct((N, ow), F32),
    )(accP, z, Wvsl, Wembl, Wgatel, Usl, Uvl, last)


def kernel(x, node_attr, edge_src, edge_dst, emb, M_proj, Wr1, br1, Wr2,
           Wvs, Wsv, Wgate, Wemb, Us, Uv):
    N = x.shape[0]
    E = edge_src.shape[0]

    M = _semi_unitary_jnp(M_proj)
    Mbig = jnp.zeros((6, 48), F32)
    MbigT = jnp.zeros((48, 6), F32)
    for i in range(2):
        for c in range(3):
            Mbig = Mbig.at[i * 3 + c, c * 16:(c + 1) * 16].set(M[i])
            MbigT = MbigT.at[c * 16:(c + 1) * 16, i * 3 + c].set(M[i])

    attr_col = node_attr.astype(F32).reshape(N, 1)
    esrc = edge_src.astype(jnp.int32)
    edst = edge_dst.astype(jnp.int32)

    nt, z = _tc_init(x, attr_col, Mbig, emb, N)

    RR = _sc_gather(nt, jnp.concatenate([esrc, edst]))
    geo = _tc_geo(RR, E)


    out = None
    for l in range(LAYERS):
        G = RR if l == 0 else _sc_gather(nt, esrc)
        m96 = _tc_edge(geo, G, Wr1[l], br1[l].reshape(1, 16), Wr2[l], E)
        accP = jax.ops.segment_sum(m96, edst, num_segments=NP)
        final = l == LAYERS - 1
        last = MbigT if final else Wsv[l + 1]
        res = _tc_node(accP, z, Wvs[l], Wemb[l], Wgate[l], Us[l], Uv[l],
                       last, N, final)
        if final:
            out = res
        else:
            nt = res
    return out
